# baseline (speedup 1.0000x reference)
import functools

import jax
import jax.numpy as jnp
from jax.experimental import pallas as pl
from jax.experimental.pallas import tpu as pltpu


def _fused(a_ref, hall_ref, ws_ref, wn_ref, bs_ref, bn_ref, gamma_ref,
           beta_ref, o_ref, rst_s, stats_s, *, tm, n, eps):
    p = pl.program_id(0)
    i = pl.program_id(1)

    @pl.when(p == 0)
    def _phase0():
        a_f = a_ref[...]
        deg = jnp.sum(a_f, axis=-1, keepdims=True)
        inv_deg = pl.reciprocal(jnp.maximum(deg, 1.0), approx=True)

        a_bf = a_f.astype(jnp.bfloat16)
        h_bf = hall_ref[...].astype(jnp.bfloat16)
        h_neigh = jnp.dot(a_bf, h_bf,
                          preferred_element_type=jnp.float32) * inv_deg

        h_tile = hall_ref[pl.ds(i * tm, tm), :].astype(jnp.bfloat16)
        x_cat = jnp.concatenate([h_tile, h_neigh.astype(jnp.bfloat16)],
                                axis=-1)
        w_cat = jnp.concatenate([ws_ref[...], wn_ref[...]], axis=0)
        rst = (jnp.dot(x_cat, w_cat.astype(jnp.bfloat16),
                       preferred_element_type=jnp.float32)
               + bs_ref[...] + bn_ref[...])
        rst = jnp.where(rst > 0, rst, 0.01 * rst)
        rst_s[pl.ds(i * tm, tm), :] = rst

        s = jnp.sum(rst, axis=0, keepdims=True)
        ss = jnp.sum(rst * rst, axis=0, keepdims=True)
        part = jnp.concatenate([s, ss], axis=0)

        @pl.when(i == 0)
        def _init():
            stats_s[...] = part

        @pl.when(i != 0)
        def _acc():
            stats_s[...] += part

    @pl.when(p == 1)
    def _phase1():
        tot = stats_s[...]
        mean = tot[0:1] * (1.0 / n)
        var = tot[1:2] * (1.0 / n) - mean * mean
        inv_std = jax.lax.rsqrt(var + eps)
        scale = gamma_ref[...] * inv_std
        shift = beta_ref[...] - mean * scale
        h_tile = hall_ref[pl.ds(i * tm, tm), :]
        o_ref[...] = h_tile + rst_s[pl.ds(i * tm, tm), :] * scale + shift


@jax.jit
def kernel(a, h, w_self, b_self, w_neigh, b_neigh, gamma, beta):
    N, F = h.shape
    tm = next(t for t in (256, 128, 64, 32, 16, 8, N) if N % t == 0)
    ntiles = N // tm
    grid = (2, ntiles)

    out = pl.pallas_call(
        functools.partial(_fused, tm=tm, n=N, eps=1e-5),
        grid=grid,
        in_specs=[
            pl.BlockSpec((tm, N),
                         lambda p, i: (jnp.where(p == 0, i, ntiles - 1), 0)),
            pl.BlockSpec((N, F), lambda p, i: (0, 0)),
            pl.BlockSpec((F, F), lambda p, i: (0, 0)),
            pl.BlockSpec((F, F), lambda p, i: (0, 0)),
            pl.BlockSpec((1, F), lambda p, i: (0, 0)),
            pl.BlockSpec((1, F), lambda p, i: (0, 0)),
            pl.BlockSpec((1, F), lambda p, i: (0, 0)),
            pl.BlockSpec((1, F), lambda p, i: (0, 0)),
        ],
        out_specs=pl.BlockSpec((tm, F),
                               lambda p, i: (jnp.where(p == 1, i, 0), 0)),
        out_shape=jax.ShapeDtypeStruct((N, F), jnp.float32),
        scratch_shapes=[
            pltpu.VMEM((N, F), jnp.float32),
            pltpu.VMEM((2, F), jnp.float32),
        ],
        compiler_params=pltpu.CompilerParams(
            dimension_semantics=("arbitrary", "arbitrary"),
            vmem_limit_bytes=100 * 1024 * 1024,
        ),
    )(a, h, w_self, w_neigh, b_self.reshape(1, F), b_neigh.reshape(1, F),
      gamma.reshape(1, F), beta.reshape(1, F))

    return out


# speedup vs baseline: 1.2217x; 1.2217x over previous
import functools

import jax
import jax.numpy as jnp
from jax.experimental import pallas as pl
from jax.experimental.pallas import tpu as pltpu


def _fused(a_ref, hall_ref, ws_ref, wn_ref, bs_ref, bn_ref, gamma_ref,
           beta_ref, o_ref, rst_s, stats_s, *, tm, n, eps):
    p = pl.program_id(0)
    i = pl.program_id(1)

    @pl.when(p == 0)
    def _phase0():
        a_f = a_ref[...]
        deg = jnp.sum(a_f, axis=-1, keepdims=True)
        inv_deg = pl.reciprocal(jnp.maximum(deg, 1.0), approx=True)

        a_bf = a_f.astype(jnp.bfloat16)
        h_bf = hall_ref[...].astype(jnp.bfloat16)
        h_neigh = jnp.dot(a_bf, h_bf,
                          preferred_element_type=jnp.float32) * inv_deg

        h_tile = hall_ref[pl.ds(i * tm, tm), :].astype(jnp.bfloat16)
        x_cat = jnp.concatenate([h_tile, h_neigh.astype(jnp.bfloat16)],
                                axis=-1)
        w_cat = jnp.concatenate([ws_ref[...], wn_ref[...]], axis=0)
        rst = (jnp.dot(x_cat, w_cat.astype(jnp.bfloat16),
                       preferred_element_type=jnp.float32)
               + bs_ref[...] + bn_ref[...])
        rst = jnp.where(rst > 0, rst, 0.01 * rst)
        rst_s[pl.ds(i * tm, tm), :] = rst

        s = jnp.sum(rst, axis=0, keepdims=True)
        ss = jnp.sum(rst * rst, axis=0, keepdims=True)
        part = jnp.concatenate([s, ss], axis=0)

        @pl.when(i == 0)
        def _init():
            stats_s[...] = part

        @pl.when(i != 0)
        def _acc():
            stats_s[...] += part

    @pl.when(p == 1)
    def _phase1():
        tot = stats_s[...]
        mean = tot[0:1] * (1.0 / n)
        var = tot[1:2] * (1.0 / n) - mean * mean
        inv_std = jax.lax.rsqrt(var + eps)
        scale = gamma_ref[...] * inv_std
        shift = beta_ref[...] - mean * scale
        h_tile = hall_ref[pl.ds(i * tm, tm), :]
        o_ref[...] = h_tile + rst_s[pl.ds(i * tm, tm), :] * scale + shift


@jax.jit
def kernel(a, h, w_self, b_self, w_neigh, b_neigh, gamma, beta):
    N, F = h.shape
    tm = next(t for t in (1024, 512, 256, 128, 64, 32, 16, 8, N) if N % t == 0)
    ntiles = N // tm
    grid = (2, ntiles)

    out = pl.pallas_call(
        functools.partial(_fused, tm=tm, n=N, eps=1e-5),
        grid=grid,
        in_specs=[
            pl.BlockSpec((tm, N),
                         lambda p, i: (jnp.where(p == 0, i, ntiles - 1), 0)),
            pl.BlockSpec((N, F), lambda p, i: (0, 0)),
            pl.BlockSpec((F, F), lambda p, i: (0, 0)),
            pl.BlockSpec((F, F), lambda p, i: (0, 0)),
            pl.BlockSpec((1, F), lambda p, i: (0, 0)),
            pl.BlockSpec((1, F), lambda p, i: (0, 0)),
            pl.BlockSpec((1, F), lambda p, i: (0, 0)),
            pl.BlockSpec((1, F), lambda p, i: (0, 0)),
        ],
        out_specs=pl.BlockSpec((tm, F),
                               lambda p, i: (jnp.where(p == 1, i, 0), 0)),
        out_shape=jax.ShapeDtypeStruct((N, F), jnp.float32),
        scratch_shapes=[
            pltpu.VMEM((N, F), jnp.float32),
            pltpu.VMEM((2, F), jnp.float32),
        ],
        compiler_params=pltpu.CompilerParams(
            dimension_semantics=("arbitrary", "arbitrary"),
            vmem_limit_bytes=100 * 1024 * 1024,
        ),
    )(a, h, w_self, w_neigh, b_self.reshape(1, F), b_neigh.reshape(1, F),
      gamma.reshape(1, F), beta.reshape(1, F))

    return out


# speedup vs baseline: 1.2306x; 1.0073x over previous
import functools

import jax
import jax.numpy as jnp
from jax.experimental import pallas as pl
from jax.experimental.pallas import tpu as pltpu


def _fused(a_ref, hall_ref, ws_ref, wn_ref, bs_ref, bn_ref, gamma_ref,
           beta_ref, o_ref, rst_s, stats_s, *, tm, n, eps):
    p = pl.program_id(0)
    i = pl.program_id(1)

    @pl.when(p == 0)
    def _phase0():
        a_f = a_ref[...]
        deg = jnp.sum(a_f, axis=-1, keepdims=True)
        inv_deg = pl.reciprocal(jnp.maximum(deg, 1.0), approx=True)

        h_neigh = jnp.dot(a_f, hall_ref[...],
                          preferred_element_type=jnp.float32) * inv_deg

        h_tile = hall_ref[pl.ds(i * tm, tm), :]
        x_cat = jnp.concatenate([h_tile, h_neigh], axis=-1)
        w_cat = jnp.concatenate([ws_ref[...], wn_ref[...]], axis=0)
        rst = (jnp.dot(x_cat, w_cat, preferred_element_type=jnp.float32)
               + bs_ref[...] + bn_ref[...])
        rst = jnp.where(rst > 0, rst, 0.01 * rst)
        rst_s[pl.ds(i * tm, tm), :] = rst

        s = jnp.sum(rst, axis=0, keepdims=True)
        ss = jnp.sum(rst * rst, axis=0, keepdims=True)
        part = jnp.concatenate([s, ss], axis=0)

        @pl.when(i == 0)
        def _init():
            stats_s[...] = part

        @pl.when(i != 0)
        def _acc():
            stats_s[...] += part

    @pl.when(p == 1)
    def _phase1():
        tot = stats_s[...]
        mean = tot[0:1] * (1.0 / n)
        var = tot[1:2] * (1.0 / n) - mean * mean
        inv_std = jax.lax.rsqrt(var + eps)
        scale = gamma_ref[...] * inv_std
        shift = beta_ref[...] - mean * scale
        h_tile = hall_ref[pl.ds(i * tm, tm), :]
        o_ref[...] = h_tile + rst_s[pl.ds(i * tm, tm), :] * scale + shift


@jax.jit
def kernel(a, h, w_self, b_self, w_neigh, b_neigh, gamma, beta):
    N, F = h.shape
    tm = next(t for t in (512, 256, 128, 64, 32, 16, 8, N) if N % t == 0)
    ntiles = N // tm
    grid = (2, ntiles)

    out = pl.pallas_call(
        functools.partial(_fused, tm=tm, n=N, eps=1e-5),
        grid=grid,
        in_specs=[
            pl.BlockSpec((tm, N),
                         lambda p, i: (jnp.where(p == 0, i, ntiles - 1), 0)),
            pl.BlockSpec((N, F), lambda p, i: (0, 0)),
            pl.BlockSpec((F, F), lambda p, i: (0, 0)),
            pl.BlockSpec((F, F), lambda p, i: (0, 0)),
            pl.BlockSpec((1, F), lambda p, i: (0, 0)),
            pl.BlockSpec((1, F), lambda p, i: (0, 0)),
            pl.BlockSpec((1, F), lambda p, i: (0, 0)),
            pl.BlockSpec((1, F), lambda p, i: (0, 0)),
        ],
        out_specs=pl.BlockSpec((tm, F),
                               lambda p, i: (jnp.where(p == 1, i, 0), 0)),
        out_shape=jax.ShapeDtypeStruct((N, F), jnp.float32),
        scratch_shapes=[
            pltpu.VMEM((N, F), jnp.float32),
            pltpu.VMEM((2, F), jnp.float32),
        ],
        compiler_params=pltpu.CompilerParams(
            dimension_semantics=("arbitrary", "arbitrary"),
            vmem_limit_bytes=100 * 1024 * 1024,
        ),
    )(a, h, w_self, w_neigh, b_self.reshape(1, F), b_neigh.reshape(1, F),
      gamma.reshape(1, F), beta.reshape(1, F))

    return out


# speedup vs baseline: 1.3130x; 1.0669x over previous
import functools

import jax
import jax.numpy as jnp
from jax.experimental import pallas as pl
from jax.experimental.pallas import tpu as pltpu


def _fused(a_ref, hall_ref, ws_ref, wn_ref, bs_ref, bn_ref, gamma_ref,
           beta_ref, o_ref, rst_s, stats_s, *, tm, ntiles, tail, n, eps):
    s = pl.program_id(0)

    @pl.when(s < ntiles)
    def _phase0():
        a_f = a_ref[...]
        deg = jnp.sum(a_f, axis=-1, keepdims=True)
        inv_deg = pl.reciprocal(jnp.maximum(deg, 1.0), approx=True)

        a_bf = a_f.astype(jnp.bfloat16)
        h_bf = hall_ref[...].astype(jnp.bfloat16)
        h_neigh = jnp.dot(a_bf, h_bf,
                          preferred_element_type=jnp.float32) * inv_deg

        h_tile = hall_ref[pl.ds(s * tm, tm), :].astype(jnp.bfloat16)
        x_cat = jnp.concatenate([h_tile, h_neigh.astype(jnp.bfloat16)],
                                axis=-1)
        w_cat = jnp.concatenate([ws_ref[...], wn_ref[...]], axis=0)
        rst = (jnp.dot(x_cat, w_cat.astype(jnp.bfloat16),
                       preferred_element_type=jnp.float32)
               + bs_ref[...] + bn_ref[...])
        rst = jnp.where(rst > 0, rst, 0.01 * rst)
        rst_s[pl.ds(s * tm, tm), :] = rst

        part = jnp.concatenate(
            [jnp.sum(rst, axis=0, keepdims=True),
             jnp.sum(rst * rst, axis=0, keepdims=True)], axis=0)

        @pl.when(s == 0)
        def _init():
            stats_s[...] = part

        @pl.when(s != 0)
        def _acc():
            stats_s[...] += part

    @pl.when(s >= ntiles)
    def _phase1():
        tot = stats_s[...]
        mean = tot[0:1] * (1.0 / n)
        var = tot[1:2] * (1.0 / n) - mean * mean
        inv_std = jax.lax.rsqrt(var + eps)
        scale = gamma_ref[...] * inv_std
        shift = beta_ref[...] - mean * scale
        j = s - ntiles
        rows = pl.ds(j * tail, tail)
        o_ref[...] = hall_ref[rows, :] + rst_s[rows, :] * scale + shift


@jax.jit
def kernel(a, h, w_self, b_self, w_neigh, b_neigh, gamma, beta):
    N, F = h.shape
    tm = next(t for t in (512, 256, 128, 64, 32, 16, 8, N) if N % t == 0)
    ntiles = N // tm
    tail_steps = 2 if N % 2 == 0 else 1
    tail = N // tail_steps
    grid = (ntiles + tail_steps,)

    out = pl.pallas_call(
        functools.partial(_fused, tm=tm, ntiles=ntiles, tail=tail,
                          n=N, eps=1e-5),
        grid=grid,
        in_specs=[
            pl.BlockSpec((tm, N),
                         lambda s: (jnp.minimum(s, ntiles - 1), 0)),
            pl.BlockSpec((N, F), lambda s: (0, 0)),
            pl.BlockSpec((F, F), lambda s: (0, 0)),
            pl.BlockSpec((F, F), lambda s: (0, 0)),
            pl.BlockSpec((1, F), lambda s: (0, 0)),
            pl.BlockSpec((1, F), lambda s: (0, 0)),
            pl.BlockSpec((1, F), lambda s: (0, 0)),
            pl.BlockSpec((1, F), lambda s: (0, 0)),
        ],
        out_specs=pl.BlockSpec(
            (tail, F),
            lambda s: (jnp.maximum(s - ntiles, 0), 0)),
        out_shape=jax.ShapeDtypeStruct((N, F), jnp.float32),
        scratch_shapes=[
            pltpu.VMEM((N, F), jnp.float32),
            pltpu.VMEM((2, F), jnp.float32),
        ],
        compiler_params=pltpu.CompilerParams(
            dimension_semantics=("arbitrary",),
            vmem_limit_bytes=100 * 1024 * 1024,
        ),
    )(a, h, w_self, w_neigh, b_self.reshape(1, F), b_neigh.reshape(1, F),
      gamma.reshape(1, F), beta.reshape(1, F))

    return out


# speedup vs baseline: 1.3259x; 1.0098x over previous
import functools

import jax
import jax.numpy as jnp
from jax.experimental import pallas as pl
from jax.experimental.pallas import tpu as pltpu


def _fused(a_ref, hall_ref, ws_ref, wn_ref, bs_ref, bn_ref, gamma_ref,
           beta_ref, o_ref, rst_s, stats_s, *, tm, ntiles, tail, n, eps):
    s = pl.program_id(0)

    @pl.when(s < ntiles)
    def _phase0():
        a_f = a_ref[...]
        deg = jnp.sum(a_f, axis=-1, keepdims=True)
        inv_deg = pl.reciprocal(jnp.maximum(deg, 1.0), approx=True)

        a_bf = a_f.astype(jnp.bfloat16)
        h_bf = hall_ref[...].astype(jnp.bfloat16)
        h_neigh = jnp.dot(a_bf, h_bf,
                          preferred_element_type=jnp.float32) * inv_deg

        h_tile = hall_ref[pl.ds(s * tm, tm), :].astype(jnp.bfloat16)
        x_cat = jnp.concatenate([h_tile, h_neigh.astype(jnp.bfloat16)],
                                axis=-1)
        w_cat = jnp.concatenate([ws_ref[...], wn_ref[...]], axis=0)
        rst = (jnp.dot(x_cat, w_cat.astype(jnp.bfloat16),
                       preferred_element_type=jnp.float32)
               + bs_ref[...] + bn_ref[...])
        rst = jnp.where(rst > 0, rst, 0.01 * rst)
        rst_s[pl.ds(s * tm, tm), :] = rst

        part = jnp.concatenate(
            [jnp.sum(rst, axis=0, keepdims=True),
             jnp.sum(rst * rst, axis=0, keepdims=True)], axis=0)

        @pl.when(s == 0)
        def _init():
            stats_s[...] = part

        @pl.when(s != 0)
        def _acc():
            stats_s[...] += part

    @pl.when(s >= ntiles)
    def _phase1():
        tot = stats_s[...]
        mean = tot[0:1] * (1.0 / n)
        var = tot[1:2] * (1.0 / n) - mean * mean
        inv_std = jax.lax.rsqrt(var + eps)
        scale = gamma_ref[...] * inv_std
        shift = beta_ref[...] - mean * scale
        j = s - ntiles
        rows = pl.ds(j * tail, tail)
        o_ref[...] = hall_ref[rows, :] + rst_s[rows, :] * scale + shift


@jax.jit
def kernel(a, h, w_self, b_self, w_neigh, b_neigh, gamma, beta):
    N, F = h.shape
    tm = next(t for t in (512, 256, 128, 64, 32, 16, 8, N) if N % t == 0)
    ntiles = N // tm
    tail_steps = 1
    tail = N // tail_steps
    grid = (ntiles + tail_steps,)

    out = pl.pallas_call(
        functools.partial(_fused, tm=tm, ntiles=ntiles, tail=tail,
                          n=N, eps=1e-5),
        grid=grid,
        in_specs=[
            pl.BlockSpec((tm, N),
                         lambda s: (jnp.minimum(s, ntiles - 1), 0)),
            pl.BlockSpec((N, F), lambda s: (0, 0)),
            pl.BlockSpec((F, F), lambda s: (0, 0)),
            pl.BlockSpec((F, F), lambda s: (0, 0)),
            pl.BlockSpec((1, F), lambda s: (0, 0)),
            pl.BlockSpec((1, F), lambda s: (0, 0)),
            pl.BlockSpec((1, F), lambda s: (0, 0)),
            pl.BlockSpec((1, F), lambda s: (0, 0)),
        ],
        out_specs=pl.BlockSpec(
            (tail, F),
            lambda s: (jnp.maximum(s - ntiles, 0), 0)),
        out_shape=jax.ShapeDtypeStruct((N, F), jnp.float32),
        scratch_shapes=[
            pltpu.VMEM((N, F), jnp.float32),
            pltpu.VMEM((2, F), jnp.float32),
        ],
        compiler_params=pltpu.CompilerParams(
            dimension_semantics=("arbitrary",),
            vmem_limit_bytes=100 * 1024 * 1024,
        ),
    )(a, h, w_self, w_neigh, b_self.reshape(1, F), b_neigh.reshape(1, F),
      gamma.reshape(1, F), beta.reshape(1, F))

    return out


# speedup vs baseline: 1.3378x; 1.0090x over previous
import functools

import jax
import jax.numpy as jnp
from jax.experimental import pallas as pl
from jax.experimental.pallas import tpu as pltpu


def _fused(a_ref, hall_ref, ws_ref, wn_ref, bs_ref, bn_ref, gamma_ref,
           beta_ref, o_ref, rst_s, stats_s, *, tm, ntiles, tail, n, eps):
    s = pl.program_id(0)

    @pl.when(s < ntiles)
    def _phase0():
        a_f = a_ref[...]
        deg = jnp.sum(a_f, axis=-1, keepdims=True)
        inv_deg = pl.reciprocal(jnp.maximum(deg, 1.0), approx=True)

        a_bf = a_f.astype(jnp.bfloat16)
        h_bf = hall_ref[...].astype(jnp.bfloat16)
        h_neigh = jnp.dot(a_bf, h_bf,
                          preferred_element_type=jnp.float32) * inv_deg

        h_tile = hall_ref[pl.ds(s * tm, tm), :].astype(jnp.bfloat16)
        x_cat = jnp.concatenate([h_tile, h_neigh.astype(jnp.bfloat16)],
                                axis=-1)
        w_cat = jnp.concatenate([ws_ref[...], wn_ref[...]], axis=0)
        rst = (jnp.dot(x_cat, w_cat.astype(jnp.bfloat16),
                       preferred_element_type=jnp.float32)
               + bs_ref[...] + bn_ref[...])
        rst = jnp.where(rst > 0, rst, 0.01 * rst)
        rst_s[pl.ds(s * tm, tm), :] = rst

        part = jnp.concatenate(
            [jnp.sum(rst, axis=0, keepdims=True),
             jnp.sum(rst * rst, axis=0, keepdims=True)], axis=0)

        @pl.when(s == 0)
        def _init():
            stats_s[...] = part

        @pl.when(s != 0)
        def _acc():
            stats_s[...] += part

    @pl.when(s >= ntiles)
    def _phase1():
        tot = stats_s[...]
        mean = tot[0:1] * (1.0 / n)
        var = tot[1:2] * (1.0 / n) - mean * mean
        inv_std = jax.lax.rsqrt(var + eps)
        scale = gamma_ref[...] * inv_std
        shift = beta_ref[...] - mean * scale
        j = s - ntiles
        rows = pl.ds(j * tail, tail)
        o_ref[...] = hall_ref[rows, :] + rst_s[rows, :] * scale + shift


@jax.jit
def kernel(a, h, w_self, b_self, w_neigh, b_neigh, gamma, beta):
    N, F = h.shape
    tm = next(t for t in (512, 256, 128, 64, 32, 16, 8, N) if N % t == 0)
    ntiles = N // tm
    tail_steps = 2 if N % 2 == 0 else 1
    tail = N // tail_steps
    grid = (ntiles + tail_steps,)

    out = pl.pallas_call(
        functools.partial(_fused, tm=tm, ntiles=ntiles, tail=tail,
                          n=N, eps=1e-5),
        grid=grid,
        in_specs=[
            pl.BlockSpec((tm, N),
                         lambda s: (jnp.minimum(s, ntiles - 1), 0)),
            pl.BlockSpec((N, F), lambda s: (0, 0)),
            pl.BlockSpec((F, F), lambda s: (0, 0)),
            pl.BlockSpec((F, F), lambda s: (0, 0)),
            pl.BlockSpec((1, F), lambda s: (0, 0)),
            pl.BlockSpec((1, F), lambda s: (0, 0)),
            pl.BlockSpec((1, F), lambda s: (0, 0)),
            pl.BlockSpec((1, F), lambda s: (0, 0)),
        ],
        out_specs=pl.BlockSpec(
            (tail, F),
            lambda s: (jnp.maximum(s - ntiles, 0), 0)),
        out_shape=jax.ShapeDtypeStruct((N, F), jnp.float32),
        scratch_shapes=[
            pltpu.VMEM((N, F), jnp.float32),
            pltpu.VMEM((2, F), jnp.float32),
        ],
        compiler_params=pltpu.CompilerParams(
            dimension_semantics=("arbitrary",),
            vmem_limit_bytes=100 * 1024 * 1024,
        ),
    )(a, h, w_self, w_neigh, b_self.reshape(1, F), b_neigh.reshape(1, F),
      gamma.reshape(1, F), beta.reshape(1, F))

    return out
